# baseline (speedup 1.0000x reference)
import functools

import jax
import jax.numpy as jnp
from jax import lax
from jax.experimental import pallas as pl
from jax.experimental.pallas import tpu as pltpu
from jax.experimental.pallas import tpu_sc as plsc

_NC, _NS = 2, 16
_NW = _NC * _NS
_CH = 128
_CPT = 80
_NWIN = 2


def _sc_mesh():
    return plsc.VectorSubcoreMesh(
        core_axis_name="c", subcore_axis_name="s",
        num_cores=_NC, num_subcores=_NS)


def _make_deg_kernel(N):
    rpt = N // _NS
    ng = _CPT // 8

    @functools.partial(
        pl.kernel,
        out_type=jax.ShapeDtypeStruct((_NC, N, 128), jnp.float32),
        mesh=_sc_mesh(),
        scratch_types=[
            pltpu.VMEM_SHARED((N, 128), jnp.float32),
            pltpu.VMEM((_CPT, _CH), jnp.int32),
            pltpu.VMEM((_CH, 128), jnp.float32),
            pltpu.SemaphoreType.DMA,
        ],
    )
    def deg_kernel(dst_hbm, zeros_hbm, ones_hbm, out_hbm, acc, didx, ones_v,
                   sem_s):
        c = lax.axis_index("c")
        s = lax.axis_index("s")
        wid = s * _NC + c
        pltpu.sync_copy(zeros_hbm.at[pl.ds(s * rpt, rpt)],
                        acc.at[pl.ds(s * rpt, rpt)])
        pltpu.sync_copy(ones_hbm, ones_v)
        pltpu.sync_copy(dst_hbm.at[pl.ds(wid * _CPT, _CPT)], didx)
        plsc.subcore_barrier()

        def body(g, carry):
            descs = [
                pltpu.async_copy(ones_v, acc.at[didx.at[g * 8 + b]], sem_s,
                                 add=True)
                for b in range(8)
            ]
            for d in descs:
                d.wait()
            return carry

        lax.fori_loop(0, ng, body, 0)
        plsc.subcore_barrier()
        pltpu.sync_copy(acc.at[pl.ds(s * rpt, rpt)],
                        out_hbm.at[c, pl.ds(s * rpt, rpt)])

    return deg_kernel


def _make_scatter_kernel(N, D):
    rpt = N // _NS
    wch = _CPT // _NWIN
    np2 = wch // 2

    @functools.partial(
        pl.kernel,
        out_type=jax.ShapeDtypeStruct((_NC, N, D), jnp.float32),
        mesh=_sc_mesh(),
        scratch_types=[
            pltpu.VMEM_SHARED((N, D), jnp.float32),
            pltpu.VMEM((wch, _CH), jnp.int32),
            pltpu.VMEM((wch, _CH), jnp.int32),
            pltpu.VMEM((2, _CH, D), jnp.float32),
            pltpu.SemaphoreType.DMA,
            pltpu.SemaphoreType.DMA,
            pltpu.SemaphoreType.DMA,
        ],
    )
    def scat_kernel(hs_hbm, src_hbm, dst_hbm, zeros_hbm, out_hbm,
                    acc, sidx, didx, rows, sem_g, sem_sa, sem_sb):
        c = lax.axis_index("c")
        s = lax.axis_index("s")
        wid = s * _NC + c
        pltpu.sync_copy(zeros_hbm.at[pl.ds(s * rpt, rpt)],
                        acc.at[pl.ds(s * rpt, rpt)])
        plsc.subcore_barrier()

        def fire_gather(g, buf):
            return pltpu.async_copy(hs_hbm.at[sidx.at[g]], rows.at[buf],
                                    sem_g)

        def drain_gather(g, buf):
            pltpu.make_async_copy(hs_hbm.at[pl.ds(0, _CH)], rows.at[buf],
                                  sem_g).wait()

        def fire_scatter(g, buf, sem):
            return pltpu.async_copy(rows.at[buf], acc.at[didx.at[g]], sem,
                                    add=True)

        def drain_scatter(g, buf, sem):
            pltpu.make_async_copy(rows.at[buf], acc.at[pl.ds(0, _CH)],
                                  sem).wait()

        def pair_body(p, carry):
            g0 = 2 * p
            g1 = 2 * p + 1
            drain_gather(g0, 0)
            sa = fire_scatter(g0, 0, sem_sa)
            @pl.when(p > 0)
            def _():
                drain_scatter(g1 - 2, 1, sem_sb)
            fire_gather(g1, 1)
            drain_gather(g1, 1)
            fire_scatter(g1, 1, sem_sb)
            sa.wait()
            @pl.when(p + 1 < np2)
            def _():
                fire_gather(g0 + 2, 0)
            return carry

        for w in range(_NWIN):
            base = wid * _CPT + w * wch
            pltpu.sync_copy(src_hbm.at[pl.ds(base, wch)], sidx)
            pltpu.sync_copy(dst_hbm.at[pl.ds(base, wch)], didx)
            fire_gather(0, 0)
            lax.fori_loop(0, np2, pair_body, 0)
            drain_scatter(wch - 1, 1, sem_sb)
        plsc.subcore_barrier()
        pltpu.sync_copy(acc.at[pl.ds(s * rpt, rpt)],
                        out_hbm.at[c, pl.ds(s * rpt, rpt)])

    return scat_kernel


def _full(shape):
    return pl.BlockSpec(shape, lambda i: (0,) * len(shape))


def _tc_mm(x, W, scale):
    N = x.shape[0]
    D = W.shape[1]

    if scale is None:
        def body(x_ref, w_ref, out_ref):
            out_ref[...] = jnp.dot(x_ref[...], w_ref[...],
                                   preferred_element_type=jnp.float32)
        args = (x, W)
    else:
        def body(x_ref, w_ref, s_ref, out_ref):
            out_ref[...] = s_ref[...] * jnp.dot(
                x_ref[...], w_ref[...], preferred_element_type=jnp.float32)
        args = (x, W, scale)

    return pl.pallas_call(
        body, out_shape=jax.ShapeDtypeStruct((N, D), jnp.float32))(*args)


def _tc_mid(sp, hs, dinv_p, b, g, be, Wn):
    N, D = hs.shape
    Dn = Wn.shape[1]

    def body(sp_ref, hs_ref, dinv_ref, b_ref, g_ref, be_ref, w_ref, out_ref):
        dinv = dinv_ref[...]
        a = dinv * (sp_ref[0] + sp_ref[1] + hs_ref[...]) + b_ref[...]
        mu = jnp.mean(a, axis=0, keepdims=True)
        var = jnp.mean(a * a, axis=0, keepdims=True) - mu * mu
        hn = g_ref[...] * (a - mu) * lax.rsqrt(var + 1e-5) + be_ref[...]
        hn = jnp.maximum(hn, 0.0)
        out_ref[...] = dinv * jnp.dot(hn, w_ref[...],
                                      preferred_element_type=jnp.float32)

    return pl.pallas_call(
        body,
        grid=(1,),
        out_shape=jax.ShapeDtypeStruct((N, Dn), jnp.float32),
        in_specs=[_full((2, N, D)), _full((N, D)), _full((N, 1)),
                  _full((1, D)), _full((1, D)), _full((1, D)),
                  _full(Wn.shape)],
        out_specs=_full((N, Dn)),
    )(sp, hs, dinv_p, b, g, be, Wn)


def _tc_fin(sp, hs, dinv_p, b, Do):
    N, D = hs.shape

    def body(sp_ref, hs_ref, dinv_ref, b_ref, out_ref):
        dinv = dinv_ref[...]
        a = dinv * (sp_ref[0] + sp_ref[1] + hs_ref[...]) + b_ref[...]
        a = a[:, :Do]
        m = jnp.max(a, axis=-1, keepdims=True)
        e = jnp.exp(a - m)
        out_ref[...] = e / jnp.sum(e, axis=-1, keepdims=True)

    return pl.pallas_call(
        body,
        grid=(1,),
        out_shape=jax.ShapeDtypeStruct((N, Do), jnp.float32),
        in_specs=[_full((2, N, D)), _full((N, D)), _full((N, 1)),
                  _full((1, D))],
        out_specs=_full((N, Do)),
    )(sp, hs, dinv_p, b)


def kernel(x, edge_index, W1, b1, g1, be1, W2, b2, g2, be2, W3, b3):
    N = x.shape[0]
    E = edge_index.shape[1]
    Dh = W1.shape[1]
    Do = W3.shape[1]
    src = edge_index[0]
    dst = edge_index[1]

    Np = ((N + 127) // 128) * 128

    Ep = _NW * _CPT * _CH
    npad = Ep - E
    pad_src = (jnp.arange(npad, dtype=jnp.int32) % N)
    pad_dst = N + (jnp.arange(npad, dtype=jnp.int32) % (Np - N))
    src2d = jnp.concatenate([src, pad_src]).reshape(_NW * _CPT, _CH)
    dst2d = jnp.concatenate([dst, pad_dst]).reshape(_NW * _CPT, _CH)

    ones128 = jnp.ones((_CH, 128), jnp.float32)
    zerosH = jnp.zeros((Np, Dh), jnp.float32)
    W3p = jnp.pad(W3, ((0, 0), (0, Dh - Do)))

    dp = _make_deg_kernel(Np)(dst2d, zerosH, ones128)
    dinv = lax.rsqrt(dp[0, :N, 0:1] + dp[1, :N, 0:1] + 1.0)

    scat = _make_scatter_kernel(Np, Dh)
    hs1 = _tc_mm(x, W1, dinv)
    sp1 = scat(hs1, src2d, dst2d, zerosH)
    hs2 = _tc_mid(sp1, hs1, dinv, b1.reshape(1, -1), g1.reshape(1, -1),
                  be1.reshape(1, -1), W2)
    sp2 = scat(hs2, src2d, dst2d, zerosH)
    hs3 = _tc_mid(sp2, hs2, dinv, b2.reshape(1, -1), g2.reshape(1, -1),
                  be2.reshape(1, -1), W3p)
    sp3 = scat(hs3, src2d, dst2d, zerosH)
    return _tc_fin(sp3, hs3, dinv, jnp.pad(b3, (0, Dh - Do)).reshape(1, -1),
                   Do)


# speedup vs baseline: 1.0023x; 1.0023x over previous
import functools

import jax
import jax.numpy as jnp
from jax import lax
from jax.experimental import pallas as pl
from jax.experimental.pallas import tpu as pltpu
from jax.experimental.pallas import tpu_sc as plsc

_NC, _NS = 2, 16
_NW = _NC * _NS
_CH = 128
_CPT = 80
_NWIN = 2


def _sc_mesh():
    return plsc.VectorSubcoreMesh(
        core_axis_name="c", subcore_axis_name="s",
        num_cores=_NC, num_subcores=_NS)


def _make_deg_kernel(N):
    rpt = N // _NS
    ng = _CPT // 8

    @functools.partial(
        pl.kernel,
        out_type=jax.ShapeDtypeStruct((_NC, N, 128), jnp.float32),
        mesh=_sc_mesh(),
        scratch_types=[
            pltpu.VMEM_SHARED((N, 128), jnp.float32),
            pltpu.VMEM((_CPT, _CH), jnp.int32),
            pltpu.VMEM((_CH, 128), jnp.float32),
            pltpu.SemaphoreType.DMA,
        ],
    )
    def deg_kernel(dst_hbm, zeros_hbm, ones_hbm, out_hbm, acc, didx, ones_v,
                   sem_s):
        c = lax.axis_index("c")
        s = lax.axis_index("s")
        wid = s * _NC + c
        pltpu.sync_copy(zeros_hbm.at[pl.ds(s * rpt, rpt)],
                        acc.at[pl.ds(s * rpt, rpt)])
        pltpu.sync_copy(ones_hbm, ones_v)
        pltpu.sync_copy(dst_hbm.at[pl.ds(wid * _CPT, _CPT)], didx)
        plsc.subcore_barrier()

        def drain8():
            for _ in range(8):
                pltpu.make_async_copy(ones_v, acc.at[pl.ds(0, _CH)],
                                      sem_s).wait()

        def body(g, carry):
            @pl.when(g > 0)
            def _():
                drain8()
            for b in range(8):
                pltpu.async_copy(ones_v, acc.at[didx.at[g * 8 + b]], sem_s,
                                 add=True)
            return carry

        lax.fori_loop(0, ng, body, 0)
        drain8()
        plsc.subcore_barrier()
        pltpu.sync_copy(acc.at[pl.ds(s * rpt, rpt)],
                        out_hbm.at[c, pl.ds(s * rpt, rpt)])

    return deg_kernel


def _make_scatter_kernel(N, D):
    rpt = N // _NS
    wch = _CPT // _NWIN
    np2 = wch // 2

    @functools.partial(
        pl.kernel,
        out_type=jax.ShapeDtypeStruct((_NC, N, D), jnp.float32),
        mesh=_sc_mesh(),
        scratch_types=[
            pltpu.VMEM_SHARED((N, D), jnp.float32),
            pltpu.VMEM((wch, _CH), jnp.int32),
            pltpu.VMEM((wch, _CH), jnp.int32),
            pltpu.VMEM((2, _CH, D), jnp.float32),
            pltpu.SemaphoreType.DMA,
            pltpu.SemaphoreType.DMA,
            pltpu.SemaphoreType.DMA,
        ],
    )
    def scat_kernel(hs_hbm, src_hbm, dst_hbm, zeros_hbm, out_hbm,
                    acc, sidx, didx, rows, sem_g, sem_sa, sem_sb):
        c = lax.axis_index("c")
        s = lax.axis_index("s")
        wid = s * _NC + c
        pltpu.sync_copy(zeros_hbm.at[pl.ds(s * rpt, rpt)],
                        acc.at[pl.ds(s * rpt, rpt)])

        def fire_gather(g, buf):
            return pltpu.async_copy(hs_hbm.at[sidx.at[g]], rows.at[buf],
                                    sem_g)

        def drain_gather(g, buf):
            pltpu.make_async_copy(hs_hbm.at[pl.ds(0, _CH)], rows.at[buf],
                                  sem_g).wait()

        def fire_scatter(g, buf, sem):
            return pltpu.async_copy(rows.at[buf], acc.at[didx.at[g]], sem,
                                    add=True)

        def drain_scatter(g, buf, sem):
            pltpu.make_async_copy(rows.at[buf], acc.at[pl.ds(0, _CH)],
                                  sem).wait()

        def pair_body(p, carry):
            g0 = 2 * p
            g1 = 2 * p + 1
            drain_gather(g0, 0)
            sa = fire_scatter(g0, 0, sem_sa)
            @pl.when(p > 0)
            def _():
                drain_scatter(g1 - 2, 1, sem_sb)
            fire_gather(g1, 1)
            drain_gather(g1, 1)
            fire_scatter(g1, 1, sem_sb)
            sa.wait()
            @pl.when(p + 1 < np2)
            def _():
                fire_gather(g0 + 2, 0)
            return carry

        for w in range(_NWIN):
            base = wid * _CPT + w * wch
            pltpu.sync_copy(src_hbm.at[pl.ds(base, wch)], sidx)
            pltpu.sync_copy(dst_hbm.at[pl.ds(base, wch)], didx)
            fire_gather(0, 0)
            if w == 0:
                plsc.subcore_barrier()
            lax.fori_loop(0, np2, pair_body, 0)
            drain_scatter(wch - 1, 1, sem_sb)
        plsc.subcore_barrier()
        pltpu.sync_copy(acc.at[pl.ds(s * rpt, rpt)],
                        out_hbm.at[c, pl.ds(s * rpt, rpt)])

    return scat_kernel


def _full(shape):
    return pl.BlockSpec(shape, lambda i: (0,) * len(shape))


def _tc_mm(x, W, scale):
    N = x.shape[0]
    D = W.shape[1]

    if scale is None:
        def body(x_ref, w_ref, out_ref):
            out_ref[...] = jnp.dot(x_ref[...], w_ref[...],
                                   preferred_element_type=jnp.float32)
        args = (x, W)
    else:
        def body(x_ref, w_ref, s_ref, out_ref):
            out_ref[...] = s_ref[...] * jnp.dot(
                x_ref[...], w_ref[...], preferred_element_type=jnp.float32)
        args = (x, W, scale)

    return pl.pallas_call(
        body, out_shape=jax.ShapeDtypeStruct((N, D), jnp.float32))(*args)


def _tc_mid(sp, hs, dinv_p, b, g, be, Wn):
    N, D = hs.shape
    Dn = Wn.shape[1]

    def body(sp_ref, hs_ref, dinv_ref, b_ref, g_ref, be_ref, w_ref, out_ref):
        dinv = dinv_ref[...]
        a = dinv * (sp_ref[0] + sp_ref[1] + hs_ref[...]) + b_ref[...]
        mu = jnp.mean(a, axis=0, keepdims=True)
        var = jnp.mean(a * a, axis=0, keepdims=True) - mu * mu
        hn = g_ref[...] * (a - mu) * lax.rsqrt(var + 1e-5) + be_ref[...]
        hn = jnp.maximum(hn, 0.0)
        out_ref[...] = dinv * jnp.dot(hn, w_ref[...],
                                      preferred_element_type=jnp.float32)

    return pl.pallas_call(
        body,
        grid=(1,),
        out_shape=jax.ShapeDtypeStruct((N, Dn), jnp.float32),
        in_specs=[_full((2, N, D)), _full((N, D)), _full((N, 1)),
                  _full((1, D)), _full((1, D)), _full((1, D)),
                  _full(Wn.shape)],
        out_specs=_full((N, Dn)),
    )(sp, hs, dinv_p, b, g, be, Wn)


def _tc_fin(sp, hs, dinv_p, b, Do):
    N, D = hs.shape

    def body(sp_ref, hs_ref, dinv_ref, b_ref, out_ref):
        dinv = dinv_ref[...]
        a = dinv * (sp_ref[0] + sp_ref[1] + hs_ref[...]) + b_ref[...]
        a = a[:, :Do]
        m = jnp.max(a, axis=-1, keepdims=True)
        e = jnp.exp(a - m)
        out_ref[...] = e / jnp.sum(e, axis=-1, keepdims=True)

    return pl.pallas_call(
        body,
        grid=(1,),
        out_shape=jax.ShapeDtypeStruct((N, Do), jnp.float32),
        in_specs=[_full((2, N, D)), _full((N, D)), _full((N, 1)),
                  _full((1, D))],
        out_specs=_full((N, Do)),
    )(sp, hs, dinv_p, b)


def kernel(x, edge_index, W1, b1, g1, be1, W2, b2, g2, be2, W3, b3):
    N = x.shape[0]
    E = edge_index.shape[1]
    Dh = W1.shape[1]
    Do = W3.shape[1]
    src = edge_index[0]
    dst = edge_index[1]

    Np = ((N + 127) // 128) * 128

    Ep = _NW * _CPT * _CH
    npad = Ep - E
    pad_src = (jnp.arange(npad, dtype=jnp.int32) % N)
    pad_dst = N + (jnp.arange(npad, dtype=jnp.int32) % (Np - N))
    src2d = jnp.concatenate([src, pad_src]).reshape(_NW * _CPT, _CH)
    dst2d = jnp.concatenate([dst, pad_dst]).reshape(_NW * _CPT, _CH)

    ones128 = jnp.ones((_CH, 128), jnp.float32)
    zerosH = jnp.zeros((Np, Dh), jnp.float32)
    W3p = jnp.pad(W3, ((0, 0), (0, Dh - Do)))

    dp = _make_deg_kernel(Np)(dst2d, zerosH, ones128)
    dinv = lax.rsqrt(dp[0, :N, 0:1] + dp[1, :N, 0:1] + 1.0)

    scat = _make_scatter_kernel(Np, Dh)
    hs1 = _tc_mm(x, W1, dinv)
    sp1 = scat(hs1, src2d, dst2d, zerosH)
    hs2 = _tc_mid(sp1, hs1, dinv, b1.reshape(1, -1), g1.reshape(1, -1),
                  be1.reshape(1, -1), W2)
    sp2 = scat(hs2, src2d, dst2d, zerosH)
    hs3 = _tc_mid(sp2, hs2, dinv, b2.reshape(1, -1), g2.reshape(1, -1),
                  be2.reshape(1, -1), W3p)
    sp3 = scat(hs3, src2d, dst2d, zerosH)
    return _tc_fin(sp3, hs3, dinv, jnp.pad(b3, (0, Dh - Do)).reshape(1, -1),
                   Do)


# speedup vs baseline: 1.0053x; 1.0029x over previous
import functools

import jax
import jax.numpy as jnp
from jax import lax
from jax.experimental import pallas as pl
from jax.experimental.pallas import tpu as pltpu
from jax.experimental.pallas import tpu_sc as plsc

_NC, _NS = 2, 16
_NW = _NC * _NS
_CH = 128
_CPT = 80
_NWIN = 2


def _sc_mesh():
    return plsc.VectorSubcoreMesh(
        core_axis_name="c", subcore_axis_name="s",
        num_cores=_NC, num_subcores=_NS)


def _make_deg_kernel(N):
    rpt = N // _NS
    ng = _CPT // 8

    @functools.partial(
        pl.kernel,
        out_type=jax.ShapeDtypeStruct((_NC, N, 128), jnp.float32),
        mesh=_sc_mesh(),
        scratch_types=[
            pltpu.VMEM_SHARED((N, 128), jnp.float32),
            pltpu.VMEM((_CPT, _CH), jnp.int32),
            pltpu.VMEM((_CH, 128), jnp.float32),
            pltpu.SemaphoreType.DMA,
        ],
    )
    def deg_kernel(dst_hbm, zeros_hbm, ones_hbm, out_hbm, acc, didx, ones_v,
                   sem_s):
        c = lax.axis_index("c")
        s = lax.axis_index("s")
        wid = s * _NC + c
        pltpu.sync_copy(zeros_hbm.at[pl.ds(s * rpt, rpt)],
                        acc.at[pl.ds(s * rpt, rpt)])
        pltpu.sync_copy(ones_hbm, ones_v)
        pltpu.sync_copy(dst_hbm.at[pl.ds(wid * _CPT, _CPT)], didx)
        plsc.subcore_barrier()

        def drain8():
            for _ in range(8):
                pltpu.make_async_copy(ones_v, acc.at[pl.ds(0, _CH)],
                                      sem_s).wait()

        def body(g, carry):
            @pl.when(g > 0)
            def _():
                drain8()
            for b in range(8):
                pltpu.async_copy(ones_v, acc.at[didx.at[g * 8 + b]], sem_s,
                                 add=True)
            return carry

        lax.fori_loop(0, ng, body, 0)
        drain8()
        plsc.subcore_barrier()
        pltpu.sync_copy(acc.at[pl.ds(s * rpt, rpt)],
                        out_hbm.at[c, pl.ds(s * rpt, rpt)])

    return deg_kernel


def _make_scatter_kernel(N, D):
    rpt = N // _NS
    wch = _CPT // _NWIN
    np2 = wch // 2

    @functools.partial(
        pl.kernel,
        out_type=jax.ShapeDtypeStruct((_NC, N, D), jnp.float32),
        mesh=_sc_mesh(),
        scratch_types=[
            pltpu.VMEM_SHARED((N, D), jnp.float32),
            pltpu.VMEM((wch, _CH), jnp.int32),
            pltpu.VMEM((wch, _CH), jnp.int32),
            pltpu.VMEM((2, _CH, D), jnp.float32),
            pltpu.SemaphoreType.DMA,
            pltpu.SemaphoreType.DMA,
            pltpu.SemaphoreType.DMA,
        ],
    )
    def scat_kernel(hs_hbm, src_hbm, dst_hbm, zeros_hbm, out_hbm,
                    acc, sidx, didx, rows, sem_g, sem_sa, sem_sb):
        c = lax.axis_index("c")
        s = lax.axis_index("s")
        wid = s * _NC + c
        pltpu.sync_copy(zeros_hbm.at[pl.ds(s * rpt, rpt)],
                        acc.at[pl.ds(s * rpt, rpt)])

        def fire_gather(g, buf):
            return pltpu.async_copy(hs_hbm.at[sidx.at[g]], rows.at[buf],
                                    sem_g, priority=1)

        def drain_gather(g, buf):
            pltpu.make_async_copy(hs_hbm.at[pl.ds(0, _CH)], rows.at[buf],
                                  sem_g).wait()

        def fire_scatter(g, buf, sem):
            return pltpu.async_copy(rows.at[buf], acc.at[didx.at[g]], sem,
                                    add=True)

        def drain_scatter(g, buf, sem):
            pltpu.make_async_copy(rows.at[buf], acc.at[pl.ds(0, _CH)],
                                  sem).wait()

        def pair_body(p, carry):
            g0 = 2 * p
            g1 = 2 * p + 1
            drain_gather(g0, 0)
            sa = fire_scatter(g0, 0, sem_sa)
            @pl.when(p > 0)
            def _():
                drain_scatter(g1 - 2, 1, sem_sb)
            fire_gather(g1, 1)
            drain_gather(g1, 1)
            fire_scatter(g1, 1, sem_sb)
            sa.wait()
            @pl.when(p + 1 < np2)
            def _():
                fire_gather(g0 + 2, 0)
            return carry

        for w in range(_NWIN):
            base = wid * _CPT + w * wch
            pltpu.sync_copy(src_hbm.at[pl.ds(base, wch)], sidx)
            pltpu.sync_copy(dst_hbm.at[pl.ds(base, wch)], didx)
            fire_gather(0, 0)
            if w == 0:
                plsc.subcore_barrier()
            lax.fori_loop(0, np2, pair_body, 0)
            drain_scatter(wch - 1, 1, sem_sb)
        plsc.subcore_barrier()
        pltpu.sync_copy(acc.at[pl.ds(s * rpt, rpt)],
                        out_hbm.at[c, pl.ds(s * rpt, rpt)])

    return scat_kernel


def _full(shape):
    return pl.BlockSpec(shape, lambda i: (0,) * len(shape))


def _tc_mm(x, W, scale):
    N = x.shape[0]
    D = W.shape[1]

    if scale is None:
        def body(x_ref, w_ref, out_ref):
            out_ref[...] = jnp.dot(x_ref[...], w_ref[...],
                                   preferred_element_type=jnp.float32)
        args = (x, W)
    else:
        def body(x_ref, w_ref, s_ref, out_ref):
            out_ref[...] = s_ref[...] * jnp.dot(
                x_ref[...], w_ref[...], preferred_element_type=jnp.float32)
        args = (x, W, scale)

    return pl.pallas_call(
        body, out_shape=jax.ShapeDtypeStruct((N, D), jnp.float32))(*args)


def _tc_mid(sp, hs, dinv_p, b, g, be, Wn):
    N, D = hs.shape
    Dn = Wn.shape[1]

    def body(sp_ref, hs_ref, dinv_ref, b_ref, g_ref, be_ref, w_ref, out_ref):
        dinv = dinv_ref[...]
        a = dinv * (sp_ref[0] + sp_ref[1] + hs_ref[...]) + b_ref[...]
        mu = jnp.mean(a, axis=0, keepdims=True)
        var = jnp.mean(a * a, axis=0, keepdims=True) - mu * mu
        hn = g_ref[...] * (a - mu) * lax.rsqrt(var + 1e-5) + be_ref[...]
        hn = jnp.maximum(hn, 0.0)
        out_ref[...] = dinv * jnp.dot(hn, w_ref[...],
                                      preferred_element_type=jnp.float32)

    return pl.pallas_call(
        body,
        grid=(1,),
        out_shape=jax.ShapeDtypeStruct((N, Dn), jnp.float32),
        in_specs=[_full((2, N, D)), _full((N, D)), _full((N, 1)),
                  _full((1, D)), _full((1, D)), _full((1, D)),
                  _full(Wn.shape)],
        out_specs=_full((N, Dn)),
    )(sp, hs, dinv_p, b, g, be, Wn)


def _tc_fin(sp, hs, dinv_p, b, Do):
    N, D = hs.shape

    def body(sp_ref, hs_ref, dinv_ref, b_ref, out_ref):
        dinv = dinv_ref[...]
        a = dinv * (sp_ref[0] + sp_ref[1] + hs_ref[...]) + b_ref[...]
        a = a[:, :Do]
        m = jnp.max(a, axis=-1, keepdims=True)
        e = jnp.exp(a - m)
        out_ref[...] = e / jnp.sum(e, axis=-1, keepdims=True)

    return pl.pallas_call(
        body,
        grid=(1,),
        out_shape=jax.ShapeDtypeStruct((N, Do), jnp.float32),
        in_specs=[_full((2, N, D)), _full((N, D)), _full((N, 1)),
                  _full((1, D))],
        out_specs=_full((N, Do)),
    )(sp, hs, dinv_p, b)


def kernel(x, edge_index, W1, b1, g1, be1, W2, b2, g2, be2, W3, b3):
    N = x.shape[0]
    E = edge_index.shape[1]
    Dh = W1.shape[1]
    Do = W3.shape[1]
    src = edge_index[0]
    dst = edge_index[1]

    Np = ((N + 127) // 128) * 128

    Ep = _NW * _CPT * _CH
    npad = Ep - E
    pad_src = (jnp.arange(npad, dtype=jnp.int32) % N)
    pad_dst = N + (jnp.arange(npad, dtype=jnp.int32) % (Np - N))
    src2d = jnp.concatenate([src, pad_src]).reshape(_NW * _CPT, _CH)
    dst2d = jnp.concatenate([dst, pad_dst]).reshape(_NW * _CPT, _CH)

    ones128 = jnp.ones((_CH, 128), jnp.float32)
    zerosH = jnp.zeros((Np, Dh), jnp.float32)
    W3p = jnp.pad(W3, ((0, 0), (0, Dh - Do)))

    dp = _make_deg_kernel(Np)(dst2d, zerosH, ones128)
    dinv = lax.rsqrt(dp[0, :N, 0:1] + dp[1, :N, 0:1] + 1.0)

    scat = _make_scatter_kernel(Np, Dh)
    hs1 = _tc_mm(x, W1, dinv)
    sp1 = scat(hs1, src2d, dst2d, zerosH)
    hs2 = _tc_mid(sp1, hs1, dinv, b1.reshape(1, -1), g1.reshape(1, -1),
                  be1.reshape(1, -1), W2)
    sp2 = scat(hs2, src2d, dst2d, zerosH)
    hs3 = _tc_mid(sp2, hs2, dinv, b2.reshape(1, -1), g2.reshape(1, -1),
                  be2.reshape(1, -1), W3p)
    sp3 = scat(hs3, src2d, dst2d, zerosH)
    return _tc_fin(sp3, hs3, dinv, jnp.pad(b3, (0, Dh - Do)).reshape(1, -1),
                   Do)
